# baseline (speedup 1.0000x reference)
import functools

import numpy as np

import jax
import jax.numpy as jnp
from jax import lax
from jax.experimental import pallas as pl
from jax.experimental.pallas import tpu as pltpu
from jax.experimental.pallas import tpu_sc as plsc

N = 10000
E = 320000
D = 128

NC = 2
NS = 16
NW = NC * NS
EW = E // NW
BLK = 80
NBLK = EW // BLK
NR = N // 16
G16 = EW // 16
ZROWS = 125

_ROWB = 1000


def _interleave_perm() -> np.ndarray:
    q = np.zeros(D, np.int64)
    for c in range(D // 32):
        for i in range(16):
            q[32 * c + 2 * i] = 32 * c + i
            q[32 * c + 2 * i + 1] = 32 * c + 16 + i
    pm = np.zeros((D, D), np.float32)
    pm[q, np.arange(D)] = 1.0
    return pm


_PM = _interleave_perm()


def _proj_body(x_ref, wp_ref, bp_ref, wl_ref, wr_ref, al_ref, ar_ref, wb_ref,
               pm_ref, hlb_ref, hr_ref, sl_ref, sr_ref, db_ref):
    h = jnp.dot(x_ref[...], wp_ref[...], preferred_element_type=jnp.float32)
    h = h + bp_ref[...]
    hl = jnp.dot(h, wl_ref[...], preferred_element_type=jnp.float32)
    hr = jnp.dot(h, wr_ref[...], preferred_element_type=jnp.float32)
    hr_ref[...] = hr
    hlb_ref[...] = jnp.dot(hl, pm_ref[...],
                           preferred_element_type=jnp.float32).astype(jnp.bfloat16)
    sl_ref[...] = jnp.sum(hl * al_ref[...], axis=1, keepdims=True)
    sr_ref[...] = jnp.sum(hr * ar_ref[...], axis=1, keepdims=True)
    hb = jnp.dot(h, wb_ref[...], preferred_element_type=jnp.float32)
    db_ref[...] = hb[:, 0:1] - hb[:, 1:2]


def _proj(x, Wp, bp2, Wl, Wr, al2, ar2, W_beta, Pm):
    grid = (N // _ROWB,)
    full = lambda shp: pl.BlockSpec(shp, lambda i: (0,) * len(shp))
    rows = lambda shp: pl.BlockSpec(shp, lambda i: (i,) + (0,) * (len(shp) - 1))
    return pl.pallas_call(
        _proj_body,
        grid=grid,
        in_specs=[
            rows((_ROWB, D)),
            full((D, D)), full((1, D)), full((D, D)), full((D, D)),
            full((1, D)), full((1, D)), full((D, 2)), full((D, D)),
        ],
        out_specs=[
            rows((_ROWB, D)), rows((_ROWB, D)),
            rows((_ROWB, 1)), rows((_ROWB, 1)), rows((_ROWB, 1)),
        ],
        out_shape=[
            jax.ShapeDtypeStruct((N, D), jnp.bfloat16),
            jax.ShapeDtypeStruct((N, D), jnp.float32),
            jax.ShapeDtypeStruct((N, 1), jnp.float32),
            jax.ShapeDtypeStruct((N, 1), jnp.float32),
            jax.ShapeDtypeStruct((N, 1), jnp.float32),
        ],
    )(x, Wp, bp2, Wl, Wr, al2, ar2, W_beta, Pm)


_MESH = plsc.VectorSubcoreMesh(core_axis_name="c", subcore_axis_name="s",
                               num_cores=NC, num_subcores=NS)
_SC_PARAMS = pltpu.CompilerParams(needs_layout_passes=False,
                                  use_tc_tiling_on_sc=False)


def _leaky(t):
    return jnp.where(t >= 0.0, t, 0.2 * t)


@functools.partial(
    pl.kernel,
    out_type=[
        jax.ShapeDtypeStruct((NW, G16, 16), jnp.float32),
        jax.ShapeDtypeStruct((NC, NR, 16), jnp.float32),
    ],
    mesh=_MESH,
    scratch_types=[
        pltpu.VMEM((G16, 16), jnp.int32),
        pltpu.VMEM((G16, 16), jnp.int32),
        pltpu.VMEM((N,), jnp.float32),
        pltpu.VMEM((N,), jnp.float32),
        pltpu.VMEM((G16, 16), jnp.float32),
        pltpu.VMEM((NR, 16), jnp.float32),
        pltpu.VMEM((5, NR // 5), jnp.int32),
        pltpu.VMEM_SHARED((NR, 16), jnp.float32),
        pltpu.SemaphoreType.DMA,
    ],
    compiler_params=_SC_PARAMS,
)
def _pass1(src_hbm, dst_hbm, sl_hbm, sr_hbm, rid_hbm, p_hbm, den_hbm,
           src_v, dst_v, sl_v, sr_v, p_v, den_v, rid_v, den_sh, sem):
    cid = lax.axis_index("c")
    sid = lax.axis_index("s")
    w = cid * NS + sid

    pltpu.sync_copy(src_hbm.at[w], src_v)
    pltpu.sync_copy(dst_hbm.at[w], dst_v)
    pltpu.sync_copy(sl_hbm, sl_v)
    pltpu.sync_copy(sr_hbm, sr_v)
    pltpu.sync_copy(rid_hbm, rid_v)

    zero16 = jnp.zeros((16,), jnp.float32)

    @pl.loop(0, NR)
    def _(r):
        den_v[r, :] = zero16

    @pl.when(sid == 0)
    def _():
        pltpu.sync_copy(den_v, den_sh)
    plsc.subcore_barrier()

    def _mstep(i, m):
        return jnp.maximum(m, sl_v[pl.ds(i * 16, 16)])
    mvec = lax.fori_loop(1, N // 16, _mstep, sl_v[pl.ds(0, 16)])
    max_sl = jnp.max(mvec)

    @pl.loop(0, G16, unroll=2)
    def _(j):
        sv = src_v[j, :]
        dv = dst_v[j, :]
        a = plsc.load_gather(sl_v, [sv])
        b = plsc.load_gather(sr_v, [dv])
        e = _leaky(a + b)
        mh = _leaky(max_sl + b)
        p = jnp.exp(e - mh)
        p_v[j, :] = p
        plsc.addupdate_scatter(den_v, [dv >> 4, dv & 15], p)

    pltpu.sync_copy(p_v, p_hbm.at[w])
    for k in range(5):
        pltpu.async_copy(den_v.at[pl.ds(k * (NR // 5), NR // 5)],
                         den_sh.at[rid_v.at[k]], sem, add=True).wait()
    plsc.subcore_barrier()

    @pl.when(sid == 0)
    def _():
        pltpu.sync_copy(den_sh, den_hbm.at[cid])


CHUNK = 25
NCHUNK = NBLK // CHUNK
RING = 4
AHEAD = RING // 2


@functools.partial(
    pl.kernel,
    out_type=jax.ShapeDtypeStruct((NC, N, D), jnp.float32),
    mesh=_MESH,
    scratch_types=[
        pltpu.VMEM((CHUNK, BLK), jnp.int32),
        pltpu.VMEM((CHUNK, BLK), jnp.int32),
        pltpu.VMEM((CHUNK, BLK), jnp.float32),
        pltpu.VMEM((2, BLK, D), jnp.bfloat16),
        pltpu.VMEM((3, BLK, D), jnp.float32),
        pltpu.VMEM_SHARED((N, D), jnp.float32),
        pltpu.SemaphoreType.DMA,
        pltpu.SemaphoreType.DMA,
    ],
    compiler_params=_SC_PARAMS,
)
def _pass2(src_hbm, dst_hbm, p_hbm, hlb_hbm, msg_hbm,
           src_v, dst_v, p_v, stg_v, rows_v, msg_sh, gsem, ssem):
    cid = lax.axis_index("c")
    sid = lax.axis_index("s")
    w = cid * NS + sid

    zero16 = jnp.zeros((16,), jnp.float32)

    for bb in range(2):
        @pl.loop(0, BLK)
        def _(r):
            for c in range(D // 16):
                rows_v[bb, r, pl.ds(c * 16, 16)] = zero16

    base_row = sid * (N // NS)
    _nz = (N // NS) // BLK
    for q in range(_nz):
        pltpu.sync_copy(rows_v.at[q & 1], msg_sh.at[pl.ds(base_row + q * BLK, BLK)])
    _tail = N // NS - _nz * BLK
    if _tail:
        pltpu.sync_copy(rows_v.at[0].at[pl.ds(0, _tail)],
                        msg_sh.at[pl.ds(base_row + _nz * BLK, _tail)])
    plsc.subcore_barrier()

    def _drain_g(sem):
        pltpu.make_async_copy(hlb_hbm.at[pl.ds(0, BLK)], stg_v.at[0], sem).wait()

    def _drain_s(sem):
        pltpu.make_async_copy(msg_hbm.at[0].at[pl.ds(0, BLK)], rows_v.at[0], sem).wait()

    for ch in range(NCHUNK):
        pltpu.sync_copy(src_hbm.at[w].at[pl.ds(ch * CHUNK, CHUNK)], src_v)
        pltpu.sync_copy(dst_hbm.at[w].at[pl.ds(ch * CHUNK, CHUNK)], dst_v)
        pltpu.sync_copy(p_hbm.at[w].at[pl.ds(ch * CHUNK, CHUNK)], p_v)

        pltpu.async_copy(hlb_hbm.at[src_v.at[0]], stg_v.at[0], gsem)

        @pl.loop(0, CHUNK)
        def _(j):
            sb = j & 1
            fb = lax.rem(j, 3)
            _drain_g(gsem)

            @pl.when(j + 1 < CHUNK)
            def _():
                pltpu.async_copy(hlb_hbm.at[src_v.at[j + 1]],
                                 stg_v.at[1 - sb], gsem)

            @pl.when(j >= 3)
            def _():
                _drain_s(ssem)

            @pl.loop(0, BLK, unroll=2)
            def _(e2):
                jfull = jnp.full((16,), j, jnp.int32)
                a = plsc.load_gather(p_v, [jfull, jnp.full((16,), e2, jnp.int32)])
                for c in range(D // 32):
                    x = stg_v[sb, e2, pl.ds(c * 32, 32)]
                    wv = plsc.bitcast(x, jnp.int32)
                    u = plsc.bitcast(wv << 16, jnp.float32)
                    v = plsc.bitcast(wv & jnp.int32(-65536), jnp.float32)
                    rows_v[fb, e2, pl.ds(c * 32, 16)] = u * a
                    rows_v[fb, e2, pl.ds(c * 32 + 16, 16)] = v * a

            pltpu.async_copy(rows_v.at[fb], msg_sh.at[dst_v.at[j]], ssem, add=True)

        for _ in range(3):
            _drain_s(ssem)

    plsc.subcore_barrier()
    pltpu.sync_copy(msg_sh.at[pl.ds(base_row, N // NS)],
                    msg_hbm.at[cid].at[pl.ds(base_row, N // NS)])


def _comb_body(msg_ref, den_ref, db_ref, hr_ref, out_ref):
    msg = msg_ref[0] + msg_ref[1]
    den = den_ref[0] + den_ref[1] + 1e-16
    b0 = jax.nn.sigmoid(db_ref[...])
    out_ref[...] = jnp.maximum(b0 * (msg / den) + (1.0 - b0) * hr_ref[...], 0.0)


def _comb(msg, den, db, hr):
    grid = (N // _ROWB,)
    return pl.pallas_call(
        _comb_body,
        grid=grid,
        in_specs=[
            pl.BlockSpec((NC, _ROWB, D), lambda i: (0, i, 0)),
            pl.BlockSpec((NC, _ROWB, 1), lambda i: (0, i, 0)),
            pl.BlockSpec((_ROWB, 1), lambda i: (i, 0)),
            pl.BlockSpec((_ROWB, D), lambda i: (i, 0)),
        ],
        out_specs=pl.BlockSpec((_ROWB, D), lambda i: (i, 0)),
        out_shape=jax.ShapeDtypeStruct((N, D), jnp.float32),
    )(msg, den, db, hr)


def kernel(x, edge_index, global_node_idx, Wp, bp, Wl, Wr, attn_l, attn_r, W_beta):
    del global_node_idx
    src1 = edge_index[0].reshape(NW, G16, 16)
    dst1 = edge_index[1].reshape(NW, G16, 16)
    src2 = edge_index[0].reshape(NW, NBLK, BLK)
    dst2 = edge_index[1].reshape(NW, NBLK, BLK)
    rid = jnp.arange(NR, dtype=jnp.int32).reshape(5, NR // 5)

    hlb, hr, sl, sr, db = _proj(x, Wp, bp.reshape(1, D), Wl, Wr,
                                attn_l.reshape(1, D), attn_r.reshape(1, D),
                                W_beta, jnp.asarray(_PM))
    p, den = _pass1(src1, dst1, sl.reshape(N), sr.reshape(N), rid)
    msg = _pass2(src2, dst2, p.reshape(NW, NBLK, BLK), hlb)
    return _comb(msg, den.reshape(NC, N, 1), db, hr)


# speedup vs baseline: 1.4816x; 1.4816x over previous
import functools

import jax
import jax.numpy as jnp
from jax import lax
from jax.experimental import pallas as pl
from jax.experimental.pallas import tpu as pltpu
from jax.experimental.pallas import tpu_sc as plsc

N = 10000
E = 320000
D = 128

NC = 2
NS = 16
NW = NC * NS
EW = E // NW
BLK = 80
NBLK = EW // BLK
NR = N // 16
G16 = EW // 16
ZROWS = 125

_ROWB = 1000


def _proj_body(x_ref, wp_ref, bp_ref, wl_ref, wr_ref, al_ref, ar_ref, wb_ref,
               hl_ref, hr_ref, sl_ref, sr_ref, db_ref):
    h = jnp.dot(x_ref[...], wp_ref[...], preferred_element_type=jnp.float32)
    h = h + bp_ref[...]
    hl = jnp.dot(h, wl_ref[...], preferred_element_type=jnp.float32)
    hr = jnp.dot(h, wr_ref[...], preferred_element_type=jnp.float32)
    hl_ref[...] = hl
    hr_ref[...] = hr
    sl_ref[...] = jnp.sum(hl * al_ref[...], axis=1, keepdims=True)
    sr_ref[...] = jnp.sum(hr * ar_ref[...], axis=1, keepdims=True)
    hb = jnp.dot(h, wb_ref[...], preferred_element_type=jnp.float32)
    db_ref[...] = hb[:, 0:1] - hb[:, 1:2]


def _proj(x, Wp, bp2, Wl, Wr, al2, ar2, W_beta):
    grid = (N // _ROWB,)
    full = lambda shp: pl.BlockSpec(shp, lambda i: (0,) * len(shp))
    rows = lambda shp: pl.BlockSpec(shp, lambda i: (i,) + (0,) * (len(shp) - 1))
    return pl.pallas_call(
        _proj_body,
        grid=grid,
        in_specs=[
            rows((_ROWB, D)),
            full((D, D)), full((1, D)), full((D, D)), full((D, D)),
            full((1, D)), full((1, D)), full((D, 2)),
        ],
        out_specs=[
            rows((_ROWB, D)), rows((_ROWB, D)),
            rows((_ROWB, 1)), rows((_ROWB, 1)), rows((_ROWB, 1)),
        ],
        out_shape=[
            jax.ShapeDtypeStruct((N, D), jnp.float32),
            jax.ShapeDtypeStruct((N, D), jnp.float32),
            jax.ShapeDtypeStruct((N, 1), jnp.float32),
            jax.ShapeDtypeStruct((N, 1), jnp.float32),
            jax.ShapeDtypeStruct((N, 1), jnp.float32),
        ],
    )(x, Wp, bp2, Wl, Wr, al2, ar2, W_beta)


_MESH = plsc.VectorSubcoreMesh(core_axis_name="c", subcore_axis_name="s",
                               num_cores=NC, num_subcores=NS)
_SC_PARAMS = pltpu.CompilerParams(needs_layout_passes=False,
                                  use_tc_tiling_on_sc=False)


def _leaky(t):
    return jnp.where(t >= 0.0, t, 0.2 * t)


@functools.partial(
    pl.kernel,
    out_type=[
        jax.ShapeDtypeStruct((NW, G16, 16), jnp.float32),
        jax.ShapeDtypeStruct((NC, NR, 16), jnp.float32),
    ],
    mesh=_MESH,
    scratch_types=[
        pltpu.VMEM((G16, 16), jnp.int32),
        pltpu.VMEM((G16, 16), jnp.int32),
        pltpu.VMEM((N,), jnp.float32),
        pltpu.VMEM((N,), jnp.float32),
        pltpu.VMEM((G16, 16), jnp.float32),
        pltpu.VMEM((NR, 16), jnp.float32),
        pltpu.VMEM((5, NR // 5), jnp.int32),
        pltpu.VMEM_SHARED((NR, 16), jnp.float32),
        pltpu.SemaphoreType.DMA,
    ],
    compiler_params=_SC_PARAMS,
)
def _pass1(src_hbm, dst_hbm, sl_hbm, sr_hbm, rid_hbm, p_hbm, den_hbm,
           src_v, dst_v, sl_v, sr_v, p_v, den_v, rid_v, den_sh, sem):
    cid = lax.axis_index("c")
    sid = lax.axis_index("s")
    w = cid * NS + sid

    pltpu.sync_copy(src_hbm.at[w], src_v)
    pltpu.sync_copy(dst_hbm.at[w], dst_v)
    pltpu.sync_copy(sl_hbm, sl_v)
    pltpu.sync_copy(sr_hbm, sr_v)
    pltpu.sync_copy(rid_hbm, rid_v)

    zero16 = jnp.zeros((16,), jnp.float32)

    @pl.loop(0, NR)
    def _(r):
        den_v[r, :] = zero16

    @pl.when(sid == 0)
    def _():
        pltpu.sync_copy(den_v, den_sh)
    plsc.subcore_barrier()

    def _mstep(i, m):
        return jnp.maximum(m, sl_v[pl.ds(i * 16, 16)])
    mvec = lax.fori_loop(1, N // 16, _mstep, sl_v[pl.ds(0, 16)])
    max_sl = jnp.max(mvec)

    @pl.loop(0, G16, unroll=2)
    def _(j):
        sv = src_v[j, :]
        dv = dst_v[j, :]
        a = plsc.load_gather(sl_v, [sv])
        b = plsc.load_gather(sr_v, [dv])
        e = _leaky(a + b)
        mh = _leaky(max_sl + b)
        p = jnp.exp(e - mh)
        p_v[j, :] = p
        plsc.addupdate_scatter(den_v, [dv >> 4, dv & 15], p)

    pltpu.sync_copy(p_v, p_hbm.at[w])
    for k in range(5):
        pltpu.async_copy(den_v.at[pl.ds(k * (NR // 5), NR // 5)],
                         den_sh.at[rid_v.at[k]], sem, add=True).wait()
    plsc.subcore_barrier()

    @pl.when(sid == 0)
    def _():
        pltpu.sync_copy(den_sh, den_hbm.at[cid])


CHUNK = 25
NCHUNK = NBLK // CHUNK
RING = 4
AHEAD = RING // 2


@functools.partial(
    pl.kernel,
    out_type=jax.ShapeDtypeStruct((NC, N, D), jnp.float32),
    mesh=_MESH,
    scratch_types=[
        pltpu.VMEM((CHUNK, BLK), jnp.int32),
        pltpu.VMEM((CHUNK, BLK), jnp.int32),
        pltpu.VMEM((CHUNK, BLK), jnp.float32),
        pltpu.VMEM((RING, BLK, D), jnp.float32),
        pltpu.VMEM_SHARED((N, D), jnp.float32),
        pltpu.SemaphoreType.DMA,
        pltpu.SemaphoreType.DMA,
    ],
    compiler_params=_SC_PARAMS,
)
def _pass2(src_hbm, dst_hbm, p_hbm, hl_hbm, msg_hbm,
           src_v, dst_v, p_v, rows_v, msg_sh, gsem, ssem):
    cid = lax.axis_index("c")
    sid = lax.axis_index("s")
    w = cid * NS + sid

    zero16 = jnp.zeros((16,), jnp.float32)

    for bb in range(2):
        @pl.loop(0, BLK)
        def _(r):
            for c in range(D // 16):
                rows_v[bb, r, pl.ds(c * 16, 16)] = zero16

    base_row = sid * (N // NS)
    _nz = (N // NS) // BLK
    for q in range(_nz):
        pltpu.sync_copy(rows_v.at[q & 1], msg_sh.at[pl.ds(base_row + q * BLK, BLK)])
    _tail = N // NS - _nz * BLK
    if _tail:
        pltpu.sync_copy(rows_v.at[0].at[pl.ds(0, _tail)],
                        msg_sh.at[pl.ds(base_row + _nz * BLK, _tail)])
    plsc.subcore_barrier()

    def _drain(sem):
        pltpu.make_async_copy(hl_hbm.at[pl.ds(0, BLK)], rows_v.at[0], sem).wait()

    for ch in range(NCHUNK):
        pltpu.sync_copy(src_hbm.at[w].at[pl.ds(ch * CHUNK, CHUNK)], src_v)
        pltpu.sync_copy(dst_hbm.at[w].at[pl.ds(ch * CHUNK, CHUNK)], dst_v)
        pltpu.sync_copy(p_hbm.at[w].at[pl.ds(ch * CHUNK, CHUNK)], p_v)

        for jj in range(AHEAD):
            pltpu.async_copy(hl_hbm.at[src_v.at[jj]], rows_v.at[jj], gsem)

        @pl.loop(0, CHUNK)
        def _(j):
            b = j & (RING - 1)
            _drain(gsem)

            @pl.when(j >= AHEAD)
            def _():
                _drain(ssem)

            @pl.when(j + AHEAD < CHUNK)
            def _():
                pltpu.async_copy(hl_hbm.at[src_v.at[j + AHEAD]],
                                 rows_v.at[(j + AHEAD) & (RING - 1)], gsem)

            @pl.loop(0, BLK, unroll=4)
            def _(e2):
                a = plsc.load_gather(
                    p_v, [jnp.full((16,), j, jnp.int32),
                          jnp.full((16,), e2, jnp.int32)])
                for c in range(D // 16):
                    slc = pl.ds(c * 16, 16)
                    rows_v[b, e2, slc] = rows_v[b, e2, slc] * a

            pltpu.async_copy(rows_v.at[b], msg_sh.at[dst_v.at[j]], ssem, add=True)

        for _ in range(AHEAD):
            _drain(ssem)

    plsc.subcore_barrier()
    pltpu.sync_copy(msg_sh.at[pl.ds(base_row, N // NS)],
                    msg_hbm.at[cid].at[pl.ds(base_row, N // NS)])


def _comb_body(msg_ref, den_ref, db_ref, hr_ref, out_ref):
    msg = msg_ref[0] + msg_ref[1]
    den = den_ref[0] + den_ref[1] + 1e-16
    b0 = jax.nn.sigmoid(db_ref[...])
    out_ref[...] = jnp.maximum(b0 * (msg / den) + (1.0 - b0) * hr_ref[...], 0.0)


def _comb(msg, den, db, hr):
    grid = (N // _ROWB,)
    return pl.pallas_call(
        _comb_body,
        grid=grid,
        in_specs=[
            pl.BlockSpec((NC, _ROWB, D), lambda i: (0, i, 0)),
            pl.BlockSpec((NC, _ROWB, 1), lambda i: (0, i, 0)),
            pl.BlockSpec((_ROWB, 1), lambda i: (i, 0)),
            pl.BlockSpec((_ROWB, D), lambda i: (i, 0)),
        ],
        out_specs=pl.BlockSpec((_ROWB, D), lambda i: (i, 0)),
        out_shape=jax.ShapeDtypeStruct((N, D), jnp.float32),
    )(msg, den, db, hr)


def kernel(x, edge_index, global_node_idx, Wp, bp, Wl, Wr, attn_l, attn_r, W_beta):
    del global_node_idx
    src1 = edge_index[0].reshape(NW, G16, 16)
    dst1 = edge_index[1].reshape(NW, G16, 16)
    src2 = edge_index[0].reshape(NW, NBLK, BLK)
    dst2 = edge_index[1].reshape(NW, NBLK, BLK)
    rid = jnp.arange(NR, dtype=jnp.int32).reshape(5, NR // 5)

    hl, hr, sl, sr, db = _proj(x, Wp, bp.reshape(1, D), Wl, Wr,
                               attn_l.reshape(1, D), attn_r.reshape(1, D), W_beta)
    p, den = _pass1(src1, dst1, sl.reshape(N), sr.reshape(N), rid)
    msg = _pass2(src2, dst2, p.reshape(NW, NBLK, BLK), hl)
    return _comb(msg, den.reshape(NC, N, 1), db, hr)


# speedup vs baseline: 1.6213x; 1.0943x over previous
import functools

import jax
import jax.numpy as jnp
from jax import lax
from jax.experimental import pallas as pl
from jax.experimental.pallas import tpu as pltpu
from jax.experimental.pallas import tpu_sc as plsc

N = 10000
E = 320000
D = 128

NC = 2
NS = 16
NW = NC * NS
EW = E // NW
BLK = 80
NBLK = EW // BLK
NR = N // 16
G16 = EW // 16
ZROWS = 125

_ROWB = 1000


def _proj_body(x_ref, wp_ref, bp_ref, wl_ref, wr_ref, al_ref, ar_ref, wb_ref,
               hl_ref, hr_ref, sl_ref, sr_ref, db_ref):
    h = jnp.dot(x_ref[...], wp_ref[...], preferred_element_type=jnp.float32)
    h = h + bp_ref[...]
    hl = jnp.dot(h, wl_ref[...], preferred_element_type=jnp.float32)
    hr = jnp.dot(h, wr_ref[...], preferred_element_type=jnp.float32)
    hl_ref[...] = hl
    hr_ref[...] = hr
    sl_ref[...] = jnp.sum(hl * al_ref[...], axis=1, keepdims=True)
    sr_ref[...] = jnp.sum(hr * ar_ref[...], axis=1, keepdims=True)
    hb = jnp.dot(h, wb_ref[...], preferred_element_type=jnp.float32)
    db_ref[...] = hb[:, 0:1] - hb[:, 1:2]


def _proj(x, Wp, bp2, Wl, Wr, al2, ar2, W_beta):
    grid = (N // _ROWB,)
    full = lambda shp: pl.BlockSpec(shp, lambda i: (0,) * len(shp))
    rows = lambda shp: pl.BlockSpec(shp, lambda i: (i,) + (0,) * (len(shp) - 1))
    return pl.pallas_call(
        _proj_body,
        grid=grid,
        in_specs=[
            rows((_ROWB, D)),
            full((D, D)), full((1, D)), full((D, D)), full((D, D)),
            full((1, D)), full((1, D)), full((D, 2)),
        ],
        out_specs=[
            rows((_ROWB, D)), rows((_ROWB, D)),
            rows((_ROWB, 1)), rows((_ROWB, 1)), rows((_ROWB, 1)),
        ],
        out_shape=[
            jax.ShapeDtypeStruct((N, D), jnp.float32),
            jax.ShapeDtypeStruct((N, D), jnp.float32),
            jax.ShapeDtypeStruct((N, 1), jnp.float32),
            jax.ShapeDtypeStruct((N, 1), jnp.float32),
            jax.ShapeDtypeStruct((N, 1), jnp.float32),
        ],
    )(x, Wp, bp2, Wl, Wr, al2, ar2, W_beta)


_MESH = plsc.VectorSubcoreMesh(core_axis_name="c", subcore_axis_name="s",
                               num_cores=NC, num_subcores=NS)
_SC_PARAMS = pltpu.CompilerParams(needs_layout_passes=False,
                                  use_tc_tiling_on_sc=False)


def _leaky(t):
    return jnp.where(t >= 0.0, t, 0.2 * t)


@functools.partial(
    pl.kernel,
    out_type=[
        jax.ShapeDtypeStruct((NW, NBLK, BLK), jnp.float32),
        jax.ShapeDtypeStruct((NC, NR, 16), jnp.float32),
    ],
    mesh=_MESH,
    scratch_types=[
        pltpu.VMEM((NBLK, BLK), jnp.int32),
        pltpu.VMEM((NBLK, BLK), jnp.int32),
        pltpu.VMEM((N,), jnp.float32),
        pltpu.VMEM((N,), jnp.float32),
        pltpu.VMEM((NBLK, BLK), jnp.float32),
        pltpu.VMEM((NR, 16), jnp.float32),
        pltpu.VMEM((5, NR // 5), jnp.int32),
        pltpu.VMEM_SHARED((NR, 16), jnp.float32),
        pltpu.SemaphoreType.DMA,
    ],
    compiler_params=_SC_PARAMS,
)
def _pass1(src_hbm, dst_hbm, sl_hbm, sr_hbm, rid_hbm, p_hbm, den_hbm,
           src_v, dst_v, sl_v, sr_v, p_v, den_v, rid_v, den_sh, sem):
    cid = lax.axis_index("c")
    sid = lax.axis_index("s")
    w = cid * NS + sid

    pltpu.sync_copy(src_hbm.at[w], src_v)
    pltpu.sync_copy(dst_hbm.at[w], dst_v)
    pltpu.sync_copy(sl_hbm, sl_v)
    pltpu.sync_copy(sr_hbm, sr_v)
    pltpu.sync_copy(rid_hbm, rid_v)

    zero16 = jnp.zeros((16,), jnp.float32)

    @pl.loop(0, NR)
    def _(r):
        den_v[r, :] = zero16

    @pl.when(sid == 0)
    def _():
        pltpu.sync_copy(den_v, den_sh)
    plsc.subcore_barrier()

    def _mstep(i, m):
        return jnp.maximum(m, sl_v[pl.ds(i * 16, 16)])
    mvec = lax.fori_loop(1, N // 16, _mstep, sl_v[pl.ds(0, 16)])
    max_sl = jnp.max(mvec)

    @pl.loop(0, NBLK)
    def _(j):
        for g in range(BLK // 16):
            sl16 = pl.ds(g * 16, 16)
            sv = src_v[j, sl16]
            dv = dst_v[j, sl16]
            a = plsc.load_gather(sl_v, [sv])
            b = plsc.load_gather(sr_v, [dv])
            e = _leaky(a + b)
            mh = _leaky(max_sl + b)
            p = jnp.exp(e - mh)
            p_v[j, sl16] = p
            plsc.addupdate_scatter(den_v, [dv >> 4, dv & 15], p)

    pltpu.sync_copy(p_v, p_hbm.at[w])
    for k in range(5):
        pltpu.async_copy(den_v.at[pl.ds(k * (NR // 5), NR // 5)],
                         den_sh.at[rid_v.at[k]], sem, add=True).wait()
    plsc.subcore_barrier()

    @pl.when(sid == 0)
    def _():
        pltpu.sync_copy(den_sh, den_hbm.at[cid])


CHUNK = 25
NCHUNK = NBLK // CHUNK
RING = 4
AHEAD = RING // 2


@functools.partial(
    pl.kernel,
    out_type=jax.ShapeDtypeStruct((NC, N, D), jnp.float32),
    mesh=_MESH,
    scratch_types=[
        pltpu.VMEM((CHUNK, BLK), jnp.int32),
        pltpu.VMEM((CHUNK, BLK), jnp.int32),
        pltpu.VMEM((CHUNK, BLK), jnp.float32),
        pltpu.VMEM((RING, BLK, D), jnp.float32),
        pltpu.VMEM_SHARED((N, D), jnp.float32),
        pltpu.SemaphoreType.DMA,
        pltpu.SemaphoreType.DMA,
    ],
    compiler_params=_SC_PARAMS,
)
def _pass2(src_hbm, dst_hbm, p_hbm, hl_hbm, msg_hbm,
           src_v, dst_v, p_v, rows_v, msg_sh, gsem, ssem):
    cid = lax.axis_index("c")
    sid = lax.axis_index("s")
    w = cid * NS + sid

    zero16 = jnp.zeros((16,), jnp.float32)

    for bb in range(2):
        @pl.loop(0, BLK)
        def _(r):
            for c in range(D // 16):
                rows_v[bb, r, pl.ds(c * 16, 16)] = zero16

    base_row = sid * (N // NS)
    _nz = (N // NS) // BLK
    for q in range(_nz):
        pltpu.sync_copy(rows_v.at[q & 1], msg_sh.at[pl.ds(base_row + q * BLK, BLK)])
    _tail = N // NS - _nz * BLK
    if _tail:
        pltpu.sync_copy(rows_v.at[0].at[pl.ds(0, _tail)],
                        msg_sh.at[pl.ds(base_row + _nz * BLK, _tail)])
    plsc.subcore_barrier()

    def _drain(sem):
        pltpu.make_async_copy(hl_hbm.at[pl.ds(0, BLK)], rows_v.at[0], sem).wait()

    for ch in range(NCHUNK):
        pltpu.sync_copy(src_hbm.at[w].at[pl.ds(ch * CHUNK, CHUNK)], src_v)
        pltpu.sync_copy(dst_hbm.at[w].at[pl.ds(ch * CHUNK, CHUNK)], dst_v)
        pltpu.sync_copy(p_hbm.at[w].at[pl.ds(ch * CHUNK, CHUNK)], p_v)

        for jj in range(AHEAD):
            pltpu.async_copy(hl_hbm.at[src_v.at[jj]], rows_v.at[jj], gsem)

        @pl.loop(0, CHUNK)
        def _(j):
            b = j & (RING - 1)
            _drain(gsem)

            @pl.when(j >= AHEAD)
            def _():
                _drain(ssem)

            @pl.when(j + AHEAD < CHUNK)
            def _():
                pltpu.async_copy(hl_hbm.at[src_v.at[j + AHEAD]],
                                 rows_v.at[(j + AHEAD) & (RING - 1)], gsem)

            @pl.loop(0, BLK, unroll=4)
            def _(e2):
                a = plsc.load_gather(
                    p_v, [jnp.full((16,), j, jnp.int32),
                          jnp.full((16,), e2, jnp.int32)])
                for c in range(D // 16):
                    slc = pl.ds(c * 16, 16)
                    rows_v[b, e2, slc] = rows_v[b, e2, slc] * a

            pltpu.async_copy(rows_v.at[b], msg_sh.at[dst_v.at[j]], ssem, add=True)

        for _ in range(AHEAD):
            _drain(ssem)

    plsc.subcore_barrier()
    pltpu.sync_copy(msg_sh.at[pl.ds(base_row, N // NS)],
                    msg_hbm.at[cid].at[pl.ds(base_row, N // NS)])


def _comb_body(msg_ref, den_ref, db_ref, hr_ref, out_ref):
    msg = msg_ref[0] + msg_ref[1]
    den = den_ref[0] + den_ref[1] + 1e-16
    b0 = jax.nn.sigmoid(db_ref[...])
    out_ref[...] = jnp.maximum(b0 * (msg / den) + (1.0 - b0) * hr_ref[...], 0.0)


def _comb(msg, den, db, hr):
    grid = (N // _ROWB,)
    return pl.pallas_call(
        _comb_body,
        grid=grid,
        in_specs=[
            pl.BlockSpec((NC, _ROWB, D), lambda i: (0, i, 0)),
            pl.BlockSpec((NC, _ROWB, 1), lambda i: (0, i, 0)),
            pl.BlockSpec((_ROWB, 1), lambda i: (i, 0)),
            pl.BlockSpec((_ROWB, D), lambda i: (i, 0)),
        ],
        out_specs=pl.BlockSpec((_ROWB, D), lambda i: (i, 0)),
        out_shape=jax.ShapeDtypeStruct((N, D), jnp.float32),
    )(msg, den, db, hr)


def kernel(x, edge_index, global_node_idx, Wp, bp, Wl, Wr, attn_l, attn_r, W_beta):
    del global_node_idx
    src = edge_index[0].reshape(NW, NBLK, BLK)
    dst = edge_index[1].reshape(NW, NBLK, BLK)
    rid = jnp.arange(NR, dtype=jnp.int32).reshape(5, NR // 5)

    hl, hr, sl, sr, db = _proj(x, Wp, bp.reshape(1, D), Wl, Wr,
                               attn_l.reshape(1, D), attn_r.reshape(1, D), W_beta)
    p, den = _pass1(src, dst, sl.reshape(N), sr.reshape(N), rid)
    msg = _pass2(src, dst, p, hl)
    return _comb(msg, den.reshape(NC, N, 1), db, hr)
